# Initial kernel scaffold; baseline (speedup 1.0000x reference)
import jax
import jax.numpy as jnp
from jax.experimental import pallas as pl


def kernel(anchor_frame, target_frame):
    raise NotImplementedError("write your pallas kernel here")


# speedup vs baseline: 14.5261x; 14.5261x over previous
import functools

import jax
import jax.numpy as jnp
from jax import lax
from jax.experimental import pallas as pl
from jax.experimental.pallas import tpu as pltpu
from jax.experimental.pallas import tpu_sc as plsc

BH = BW = 24
BLK = 16
S = 3
NB = BH * BW
FEAT = 8 * 3 * BLK * BLK
ROWS = 8 * 3 * 384 * 24
NC_SC = 2
NS_SC = 16
NW = NC_SC * NS_SC
RPW = ROWS // NW
CHUNK = 128
NCH = RPW // CHUNK


KCH = 1024
KSTEPS = FEAT // KCH


def _tc_cost_body(a_ref, t_ref, idx_ref, g_acc, nt_acc):
    k = pl.program_id(0)

    @pl.when(k == 0)
    def _init():
        g_acc[...] = jnp.zeros_like(g_acc)
        nt_acc[...] = jnp.zeros_like(nt_acc)

    A = a_ref[...]
    T = t_ref[...]
    g_acc[...] += lax.dot_general(
        A, T, (((1,), (1,)), ((), ())),
        preferred_element_type=jnp.float32,
        precision=lax.Precision.HIGHEST)
    ones = jnp.ones((1, KCH), jnp.float32)
    nt_acc[...] += lax.dot_general(
        ones, T * T, (((1,), (1,)), ((), ())),
        preferred_element_type=jnp.float32,
        precision=lax.Precision.HIGHEST)

    @pl.when(k == KSTEPS - 1)
    def _epilogue():
        _tc_epilogue(idx_ref, g_acc, nt_acc)


def _tc_epilogue(idx_ref, g_acc, nt_acc):
    G = g_acc[...]
    nT = nt_acc[...]
    i_io = lax.broadcasted_iota(jnp.int32, (NB, NB), 0)
    j_io = lax.broadcasted_iota(jnp.int32, (NB, NB), 1)
    o = j_io - i_io
    dy = (o + 12) // 24
    dx = o - 24 * dy
    by = i_io // BW
    bx = i_io % BW
    valid = ((dy >= -S) & (dy <= S) & (dx >= -S) & (dx <= S)
             & (by + dy >= 0) & (by + dy < BH)
             & (bx + dx >= 0) & (bx + dx < BW))
    cost = nT - 2.0 * G
    cost = jnp.where(valid, cost, jnp.inf)
    cost3 = cost.reshape(BH, BW, NB)
    j3 = lax.broadcasted_iota(jnp.int32, (BH, BW, NB), 2)
    m = jnp.min(cost3, axis=2, keepdims=True)
    bj = jnp.min(jnp.where(cost3 <= m, j3, jnp.int32(1 << 30)), axis=2)
    ny = bj // BW
    nx = bj % BW
    u = 384 * ny + nx
    nc_io = lax.broadcasted_iota(jnp.int32, (24, BH, BLK, BW), 0)
    r_io = lax.broadcasted_iota(jnp.int32, (24, BH, BLK, BW), 2)
    idx_ref[...] = 9216 * nc_io + 24 * r_io + u[None, :, None, :]


def _tc_cost(A_b, T_b):
    return pl.pallas_call(
        _tc_cost_body,
        grid=(KSTEPS,),
        in_specs=[
            pl.BlockSpec((NB, KCH), lambda k: (0, k)),
            pl.BlockSpec((NB, KCH), lambda k: (0, k)),
        ],
        out_specs=pl.BlockSpec((24, BH, BLK, BW), lambda k: (0, 0, 0, 0)),
        out_shape=jax.ShapeDtypeStruct((24, BH, BLK, BW), jnp.int32),
        scratch_shapes=[
            pltpu.VMEM((NB, NB), jnp.float32),
            pltpu.VMEM((1, NB), jnp.float32),
        ],
    )(A_b, T_b)


def _sc_gather_body(table_hbm, idx_hbm, out_hbm, idx_v, rows_v, sem):
    wid = lax.axis_index("s") * NC_SC + lax.axis_index("c")
    base = wid * RPW
    pltpu.sync_copy(idx_hbm.at[wid], idx_v)

    def chunk(g):
        pltpu.async_copy(table_hbm.at[idx_v.at[g]],
                         rows_v.at[pl.ds(g * CHUNK, CHUNK)], sem)

    pl.loop(0, NCH)(chunk)
    pltpu.make_async_copy(table_hbm.at[pl.ds(0, RPW)], rows_v, sem).wait()
    pltpu.sync_copy(rows_v, out_hbm.at[pl.ds(base, RPW)])


@functools.lru_cache(maxsize=1)
def _sc_gather_fn():
    return pl.kernel(
        _sc_gather_body,
        out_type=jax.ShapeDtypeStruct((ROWS, 16), jnp.float32),
        mesh=plsc.VectorSubcoreMesh(core_axis_name="c", subcore_axis_name="s"),
        scratch_types=[
            pltpu.VMEM((NCH, CHUNK), jnp.int32),
            pltpu.VMEM((RPW, 16), jnp.float32),
            pltpu.SemaphoreType.DMA,
        ],
        compiler_params=pltpu.CompilerParams(use_tc_tiling_on_sc=False),
    )


def _blockify(f):
    N, C, H, W = f.shape
    return f.reshape(N, C, H // BLK, BLK, W // BLK, BLK).transpose(2, 4, 0, 1, 3, 5)


def kernel(anchor_frame, target_frame):
    A_b = _blockify(anchor_frame).reshape(NB, FEAT)
    T_b = _blockify(target_frame).reshape(NB, FEAT)
    idx4 = _tc_cost(A_b, T_b)
    idx2d = idx4.reshape(NW, NCH, CHUNK)
    table = anchor_frame.reshape(ROWS, 16)
    out2d = _sc_gather_fn()(table, idx2d)
    return out2d.reshape(8, 3, 384, 384)


# speedup vs baseline: 29.8810x; 2.0571x over previous
import functools

import jax
import jax.numpy as jnp
from jax import lax
from jax.experimental import pallas as pl
from jax.experimental.pallas import tpu as pltpu
from jax.experimental.pallas import tpu_sc as plsc

BH = BW = 24
BLK = 16
S = 3
NB = BH * BW
FEAT = 8 * 3 * BLK * BLK
ROWS = 8 * 3 * 384 * 24
NC_SC = 2
NS_SC = 16
NW = NC_SC * NS_SC
RPW = ROWS // NW
CHUNK = 128
NCH = RPW // CHUNK


KSUB = 8
KSTEPS = FEAT // (KSUB * 128)


def _tc_cost_body(a_ref, t_ref, idx_ref, g_acc, nt_acc):
    k = pl.program_id(0)

    @pl.when(k == 0)
    def _init():
        g_acc[...] = jnp.zeros_like(g_acc)
        nt_acc[...] = jnp.zeros_like(nt_acc)

    for t in range(KSUB):
        A = a_ref[:, t, :]
        T = t_ref[:, t, :]
        g_acc[...] += lax.dot_general(
            A, T, (((1,), (1,)), ((), ())),
            preferred_element_type=jnp.float32,
            precision=lax.Precision.HIGHEST)
        ones = jnp.ones((1, 128), jnp.float32)
        nt_acc[...] += lax.dot_general(
            ones, T * T, (((1,), (1,)), ((), ())),
            preferred_element_type=jnp.float32,
            precision=lax.Precision.HIGHEST)

    @pl.when(k == KSTEPS - 1)
    def _epilogue():
        _tc_epilogue(idx_ref, g_acc, nt_acc)


def _tc_epilogue(idx_ref, g_acc, nt_acc):
    G = g_acc[...]
    nT = nt_acc[...]
    i_io = lax.broadcasted_iota(jnp.int32, (NB, NB), 0)
    j_io = lax.broadcasted_iota(jnp.int32, (NB, NB), 1)
    o = j_io - i_io
    dy = (o + 12) // 24
    dx = o - 24 * dy
    by = i_io // BW
    bx = i_io % BW
    valid = ((dy >= -S) & (dy <= S) & (dx >= -S) & (dx <= S)
             & (by + dy >= 0) & (by + dy < BH)
             & (bx + dx >= 0) & (bx + dx < BW))
    cost = nT - 2.0 * G
    cost = jnp.where(valid, cost, jnp.inf)
    cost3 = cost.reshape(BH, BW, NB)
    j3 = lax.broadcasted_iota(jnp.int32, (BH, BW, NB), 2)
    m = jnp.min(cost3, axis=2, keepdims=True)
    bj = jnp.min(jnp.where(cost3 <= m, j3, jnp.int32(1 << 30)), axis=2)
    ny = bj // BW
    nx = bj % BW
    u = 384 * ny + nx
    nc_io = lax.broadcasted_iota(jnp.int32, (24, BH, BLK, BW), 0)
    r_io = lax.broadcasted_iota(jnp.int32, (24, BH, BLK, BW), 2)
    idx_ref[...] = 9216 * nc_io + 24 * r_io + u[None, :, None, :]


def _tc_cost(A_3d, T_3d):
    return pl.pallas_call(
        _tc_cost_body,
        grid=(KSTEPS,),
        in_specs=[
            pl.BlockSpec((NB, KSUB, 128), lambda k: (0, k, 0)),
            pl.BlockSpec((NB, KSUB, 128), lambda k: (0, k, 0)),
        ],
        out_specs=pl.BlockSpec((24, BH, BLK, BW), lambda k: (0, 0, 0, 0)),
        out_shape=jax.ShapeDtypeStruct((24, BH, BLK, BW), jnp.int32),
        scratch_shapes=[
            pltpu.VMEM((NB, NB), jnp.float32),
            pltpu.VMEM((1, NB), jnp.float32),
        ],
    )(A_3d, T_3d)


def _sc_blockify_body(a_tab, t_tab, a_out, t_out, idx_v, rows_v, sem):
    wid = lax.axis_index("s") * NC_SC + lax.axis_index("c")
    base = wid * RPW
    lane24 = 24 * lax.iota(jnp.int32, 16)

    def idx_step(ii):
        nc = ii % 24
        i_blk = wid * 18 + ii // 24
        sbase = 9216 * nc + 384 * (i_blk // 24) + i_blk % 24
        idx_v[ii // 8, pl.ds(16 * (ii % 8), 16)] = sbase + lane24

    pl.loop(0, 432)(idx_step)

    def run_gather(tab, out):
        def chunk(g):
            pltpu.async_copy(tab.at[idx_v.at[g]],
                             rows_v.at[pl.ds(g * CHUNK, CHUNK)], sem)

        pl.loop(0, NCH)(chunk)
        pltpu.make_async_copy(tab.at[pl.ds(0, RPW)], rows_v, sem).wait()
        pltpu.sync_copy(rows_v, out.at[pl.ds(base, RPW)])

    run_gather(a_tab, a_out)
    run_gather(t_tab, t_out)


@functools.lru_cache(maxsize=1)
def _sc_blockify_fn():
    return pl.kernel(
        _sc_blockify_body,
        out_type=(jax.ShapeDtypeStruct((ROWS, 16), jnp.float32),
                  jax.ShapeDtypeStruct((ROWS, 16), jnp.float32)),
        mesh=plsc.VectorSubcoreMesh(core_axis_name="c", subcore_axis_name="s"),
        scratch_types=[
            pltpu.VMEM((NCH, CHUNK), jnp.int32),
            pltpu.VMEM((RPW, 16), jnp.float32),
            pltpu.SemaphoreType.DMA,
        ],
        compiler_params=pltpu.CompilerParams(use_tc_tiling_on_sc=False),
    )


def _sc_gather_body(table_hbm, idx_hbm, out_hbm, idx_v, rows_v, sem):
    wid = lax.axis_index("s") * NC_SC + lax.axis_index("c")
    base = wid * RPW
    pltpu.sync_copy(idx_hbm.at[wid], idx_v)

    def chunk(g):
        pltpu.async_copy(table_hbm.at[idx_v.at[g]],
                         rows_v.at[pl.ds(g * CHUNK, CHUNK)], sem)

    pl.loop(0, NCH)(chunk)
    pltpu.make_async_copy(table_hbm.at[pl.ds(0, RPW)], rows_v, sem).wait()
    pltpu.sync_copy(rows_v, out_hbm.at[pl.ds(base, RPW)])


@functools.lru_cache(maxsize=1)
def _sc_gather_fn():
    return pl.kernel(
        _sc_gather_body,
        out_type=jax.ShapeDtypeStruct((ROWS, 16), jnp.float32),
        mesh=plsc.VectorSubcoreMesh(core_axis_name="c", subcore_axis_name="s"),
        scratch_types=[
            pltpu.VMEM((NCH, CHUNK), jnp.int32),
            pltpu.VMEM((RPW, 16), jnp.float32),
            pltpu.SemaphoreType.DMA,
        ],
        compiler_params=pltpu.CompilerParams(use_tc_tiling_on_sc=False),
    )


def kernel(anchor_frame, target_frame):
    a_tab = anchor_frame.reshape(ROWS, 16)
    t_tab = target_frame.reshape(ROWS, 16)
    A_2d, T_2d = _sc_blockify_fn()(a_tab, t_tab)
    idx4 = _tc_cost(A_2d.reshape(NB, 48, 128), T_2d.reshape(NB, 48, 128))
    idx2d = idx4.reshape(NW, NCH, CHUNK)
    out2d = _sc_gather_fn()(a_tab, idx2d)
    return out2d.reshape(8, 3, 384, 384)


# speedup vs baseline: 30.4836x; 1.0202x over previous
import functools

import jax
import jax.numpy as jnp
from jax import lax
from jax.experimental import pallas as pl
from jax.experimental.pallas import tpu as pltpu
from jax.experimental.pallas import tpu_sc as plsc

BH = BW = 24
BLK = 16
S = 3
NB = BH * BW
FEAT = 8 * 3 * BLK * BLK
ROWS = 8 * 3 * 384 * 24
NC_SC = 2
NS_SC = 16
NW = NC_SC * NS_SC
RPW = ROWS // NW
CHUNK = 128
NCH = RPW // CHUNK


KSUB = 8
KSTEPS = FEAT // (KSUB * 128)


def _tc_cost_body(a_ref, t_ref, idx_ref, g_acc, nt_acc):
    k = pl.program_id(0)

    @pl.when(k == 0)
    def _init():
        g_acc[...] = jnp.zeros_like(g_acc)
        nt_acc[...] = jnp.zeros_like(nt_acc)

    def f32_dot(x, y):
        return lax.dot_general(x, y, (((1,), (1,)), ((), ())),
                               preferred_element_type=jnp.float32,
                               precision=lax.Precision.HIGHEST)

    g_step = f32_dot(a_ref[:, 0, :], t_ref[:, 0, :])
    nt_step = f32_dot(jnp.ones((1, 128), jnp.float32),
                      t_ref[:, 0, :] * t_ref[:, 0, :])
    for t in range(1, KSUB):
        T = t_ref[:, t, :]
        g_step += f32_dot(a_ref[:, t, :], T)
        nt_step += f32_dot(jnp.ones((1, 128), jnp.float32), T * T)
    g_acc[...] += g_step
    nt_acc[...] += nt_step

    @pl.when(k == KSTEPS - 1)
    def _epilogue():
        _tc_epilogue(idx_ref, g_acc, nt_acc)


def _tc_epilogue(idx_ref, g_acc, nt_acc):
    G = g_acc[...]
    nT = nt_acc[...]
    i_io = lax.broadcasted_iota(jnp.int32, (NB, NB), 0)
    j_io = lax.broadcasted_iota(jnp.int32, (NB, NB), 1)
    o = j_io - i_io
    dy = (o + 12) // 24
    dx = o - 24 * dy
    by = i_io // BW
    bx = i_io % BW
    valid = ((dy >= -S) & (dy <= S) & (dx >= -S) & (dx <= S)
             & (by + dy >= 0) & (by + dy < BH)
             & (bx + dx >= 0) & (bx + dx < BW))
    cost = nT - 2.0 * G
    cost = jnp.where(valid, cost, jnp.inf)
    cost3 = cost.reshape(BH, BW, NB)
    j3 = lax.broadcasted_iota(jnp.int32, (BH, BW, NB), 2)
    m = jnp.min(cost3, axis=2, keepdims=True)
    bj = jnp.min(jnp.where(cost3 <= m, j3, jnp.int32(1 << 30)), axis=2)
    ny = bj // BW
    nx = bj % BW
    u = 384 * ny + nx
    nc_io = lax.broadcasted_iota(jnp.int32, (24, BH, BLK, BW), 0)
    r_io = lax.broadcasted_iota(jnp.int32, (24, BH, BLK, BW), 2)
    idx_ref[...] = 9216 * nc_io + 24 * r_io + u[None, :, None, :]


def _tc_cost(A_3d, T_3d):
    return pl.pallas_call(
        _tc_cost_body,
        grid=(KSTEPS,),
        in_specs=[
            pl.BlockSpec((NB, KSUB, 128), lambda k: (0, k, 0)),
            pl.BlockSpec((NB, KSUB, 128), lambda k: (0, k, 0)),
        ],
        out_specs=pl.BlockSpec((24, BH, BLK, BW), lambda k: (0, 0, 0, 0)),
        out_shape=jax.ShapeDtypeStruct((24, BH, BLK, BW), jnp.int32),
        scratch_shapes=[
            pltpu.VMEM((NB, NB), jnp.float32),
            pltpu.VMEM((1, NB), jnp.float32),
        ],
    )(A_3d, T_3d)


def _sc_blockify_body(a_tab, t_tab, a_out, t_out, idx_v, rows_v, sem):
    wid = lax.axis_index("s") * NC_SC + lax.axis_index("c")
    base = wid * RPW
    lane24 = 24 * lax.iota(jnp.int32, 16)

    def idx_step(ii):
        nc = ii % 24
        i_blk = wid * 18 + ii // 24
        sbase = 9216 * nc + 384 * (i_blk // 24) + i_blk % 24
        idx_v[ii // 8, pl.ds(16 * (ii % 8), 16)] = sbase + lane24

    pl.loop(0, 432)(idx_step)

    def run_gather(tab, out):
        def chunk(g):
            pltpu.async_copy(tab.at[idx_v.at[g]],
                             rows_v.at[pl.ds(g * CHUNK, CHUNK)], sem)

        pl.loop(0, NCH)(chunk)
        pltpu.make_async_copy(tab.at[pl.ds(0, RPW)], rows_v, sem).wait()
        pltpu.sync_copy(rows_v, out.at[pl.ds(base, RPW)])

    run_gather(a_tab, a_out)
    run_gather(t_tab, t_out)


@functools.lru_cache(maxsize=1)
def _sc_blockify_fn():
    return pl.kernel(
        _sc_blockify_body,
        out_type=(jax.ShapeDtypeStruct((ROWS, 16), jnp.float32),
                  jax.ShapeDtypeStruct((ROWS, 16), jnp.float32)),
        mesh=plsc.VectorSubcoreMesh(core_axis_name="c", subcore_axis_name="s"),
        scratch_types=[
            pltpu.VMEM((NCH, CHUNK), jnp.int32),
            pltpu.VMEM((RPW, 16), jnp.float32),
            pltpu.SemaphoreType.DMA,
        ],
        compiler_params=pltpu.CompilerParams(use_tc_tiling_on_sc=False),
    )


def _sc_gather_body(table_hbm, idx_hbm, out_hbm, idx_v, rows_v, sem):
    wid = lax.axis_index("s") * NC_SC + lax.axis_index("c")
    base = wid * RPW
    pltpu.sync_copy(idx_hbm.at[wid], idx_v)

    def chunk(g):
        pltpu.async_copy(table_hbm.at[idx_v.at[g]],
                         rows_v.at[pl.ds(g * CHUNK, CHUNK)], sem)

    pl.loop(0, NCH)(chunk)
    pltpu.make_async_copy(table_hbm.at[pl.ds(0, RPW)], rows_v, sem).wait()
    pltpu.sync_copy(rows_v, out_hbm.at[pl.ds(base, RPW)])


@functools.lru_cache(maxsize=1)
def _sc_gather_fn():
    return pl.kernel(
        _sc_gather_body,
        out_type=jax.ShapeDtypeStruct((ROWS, 16), jnp.float32),
        mesh=plsc.VectorSubcoreMesh(core_axis_name="c", subcore_axis_name="s"),
        scratch_types=[
            pltpu.VMEM((NCH, CHUNK), jnp.int32),
            pltpu.VMEM((RPW, 16), jnp.float32),
            pltpu.SemaphoreType.DMA,
        ],
        compiler_params=pltpu.CompilerParams(use_tc_tiling_on_sc=False),
    )


def kernel(anchor_frame, target_frame):
    a_tab = anchor_frame.reshape(ROWS, 16)
    t_tab = target_frame.reshape(ROWS, 16)
    A_2d, T_2d = _sc_blockify_fn()(a_tab, t_tab)
    idx4 = _tc_cost(A_2d.reshape(NB, 48, 128), T_2d.reshape(NB, 48, 128))
    idx2d = idx4.reshape(NW, NCH, CHUNK)
    out2d = _sc_gather_fn()(a_tab, idx2d)
    return out2d.reshape(8, 3, 384, 384)
